# baseline (speedup 1.0000x reference)
import jax
import jax.numpy as jnp
from jax.experimental import pallas as pl
from jax.experimental.pallas import tpu as pltpu

_N = 4096
_MAXDEG = 64
_SIZES = (256,) * 16
_OFFS = []
_o = 0
for _s in _SIZES:
    _OFFS.append(_o)
    _o += _s
_OFFS = tuple(_OFFS)
_NB = len(_SIZES)
_BMAX = 256
_RING = 4


def _stream_kernel(rank_ref, c_ref, mask_hbm, out_hbm,
                   in0, in1, in2, in3, ob0, ob1, ob2, ob3, g2d,
                   sin0, sin1, sin2, sin3, sout0, sout1, sout2, sout3):
    in_bufs = (in0, in1, in2, in3)
    out_bufs = (ob0, ob1, ob2, ob3)
    sin = (sin0, sin1, sin2, sin3)
    sout = (sout0, sout1, sout2, sout3)

    def in_copy(k):
        b = k % _RING
        return pltpu.make_async_copy(
            mask_hbm.at[pl.ds(_OFFS[k], _SIZES[k]), :],
            in_bufs[b].at[pl.ds(0, _SIZES[k]), :],
            sin[b])

    def out_copy(k):
        b = k % _RING
        return pltpu.make_async_copy(
            out_bufs[b].at[pl.ds(0, _SIZES[k]), :],
            out_hbm.at[pl.ds(_OFFS[k], _SIZES[k]), :],
            sout[b])

    for _k in range(_RING):
        in_copy(_k).start()

    rc = jnp.minimum(rank_ref[...], _MAXDEG - 1)
    bits = [((rc >> b) & 1) == 1 for b in range(6)]
    vals = [c_ref[k] for k in range(_MAXDEG)]
    for b in range(6):
        vals = [jnp.where(bits[b], hi, lo)
                for lo, hi in zip(vals[0::2], vals[1::2])]
    g2d[...] = vals[0][:, None]

    for k in range(_NB):
        b = k % _RING
        in_copy(k).wait()
        if k >= _RING:
            out_copy(k - _RING).wait()
        gs = g2d[pl.ds(_OFFS[k], _SIZES[k]), :]
        out_bufs[b][pl.ds(0, _SIZES[k]), :] = (
            gs * in_bufs[b][pl.ds(0, _SIZES[k]), :])
        out_copy(k).start()
        if k + _RING < _NB:
            in_copy(k + _RING).start()

    for _k in range(_NB - _RING, _NB):
        out_copy(_k).wait()


def kernel(x, rank, sparse_mask, c):
    del x
    return pl.pallas_call(
        _stream_kernel,
        in_specs=[
            pl.BlockSpec(memory_space=pltpu.VMEM),
            pl.BlockSpec(memory_space=pltpu.SMEM),
            pl.BlockSpec(memory_space=pl.ANY),
        ],
        out_specs=pl.BlockSpec(memory_space=pl.ANY),
        out_shape=jax.ShapeDtypeStruct((_N, _N), jnp.float32),
        scratch_shapes=[
            pltpu.VMEM((_BMAX, _N), jnp.float32),
            pltpu.VMEM((_BMAX, _N), jnp.float32),
            pltpu.VMEM((_BMAX, _N), jnp.float32),
            pltpu.VMEM((_BMAX, _N), jnp.float32),
            pltpu.VMEM((_BMAX, _N), jnp.float32),
            pltpu.VMEM((_BMAX, _N), jnp.float32),
            pltpu.VMEM((_BMAX, _N), jnp.float32),
            pltpu.VMEM((_BMAX, _N), jnp.float32),
            pltpu.VMEM((_N, 1), jnp.float32),
            pltpu.SemaphoreType.DMA,
            pltpu.SemaphoreType.DMA,
            pltpu.SemaphoreType.DMA,
            pltpu.SemaphoreType.DMA,
            pltpu.SemaphoreType.DMA,
            pltpu.SemaphoreType.DMA,
            pltpu.SemaphoreType.DMA,
            pltpu.SemaphoreType.DMA,
        ],
    )(rank, c, sparse_mask)


# speedup vs baseline: 1.0068x; 1.0068x over previous
import jax
import jax.numpy as jnp
from jax.experimental import pallas as pl
from jax.experimental.pallas import tpu as pltpu

_N = 4096
_MAXDEG = 64
_SIZES = (512,) * 8
_OFFS = []
_o = 0
for _s in _SIZES:
    _OFFS.append(_o)
    _o += _s
_OFFS = tuple(_OFFS)
_NB = len(_SIZES)
_BMAX = 512
_RING = 3


def _stream_kernel(rank_ref, c_ref, mask_hbm, out_hbm,
                   in0, in1, in2, ob0, ob1, ob2, g2d,
                   sin0, sin1, sin2, sout0, sout1, sout2):
    in_bufs = (in0, in1, in2)
    out_bufs = (ob0, ob1, ob2)
    sin = (sin0, sin1, sin2)
    sout = (sout0, sout1, sout2)

    def in_copy(k):
        b = k % _RING
        return pltpu.make_async_copy(
            mask_hbm.at[pl.ds(_OFFS[k], _SIZES[k]), :],
            in_bufs[b].at[pl.ds(0, _SIZES[k]), :],
            sin[b])

    def out_copy(k):
        b = k % _RING
        return pltpu.make_async_copy(
            out_bufs[b].at[pl.ds(0, _SIZES[k]), :],
            out_hbm.at[pl.ds(_OFFS[k], _SIZES[k]), :],
            sout[b])

    for _k in range(_RING):
        in_copy(_k).start()

    rc = jnp.minimum(rank_ref[...], _MAXDEG - 1)
    bits = [((rc >> b) & 1) == 1 for b in range(6)]
    vals = [c_ref[k] for k in range(_MAXDEG)]
    for b in range(6):
        vals = [jnp.where(bits[b], hi, lo)
                for lo, hi in zip(vals[0::2], vals[1::2])]
    g2d[...] = vals[0][:, None]

    for k in range(_NB):
        b = k % _RING
        in_copy(k).wait()
        if k >= _RING:
            out_copy(k - _RING).wait()
        gs = g2d[pl.ds(_OFFS[k], _SIZES[k]), :]
        out_bufs[b][pl.ds(0, _SIZES[k]), :] = (
            gs * in_bufs[b][pl.ds(0, _SIZES[k]), :])
        out_copy(k).start()
        if k + _RING < _NB:
            in_copy(k + _RING).start()

    for _k in range(_NB - _RING, _NB):
        out_copy(_k).wait()


def kernel(x, rank, sparse_mask, c):
    del x
    return pl.pallas_call(
        _stream_kernel,
        in_specs=[
            pl.BlockSpec(memory_space=pltpu.VMEM),
            pl.BlockSpec(memory_space=pltpu.SMEM),
            pl.BlockSpec(memory_space=pl.ANY),
        ],
        out_specs=pl.BlockSpec(memory_space=pl.ANY),
        out_shape=jax.ShapeDtypeStruct((_N, _N), jnp.float32),
        scratch_shapes=[
            pltpu.VMEM((_BMAX, _N), jnp.float32),
            pltpu.VMEM((_BMAX, _N), jnp.float32),
            pltpu.VMEM((_BMAX, _N), jnp.float32),
            pltpu.VMEM((_BMAX, _N), jnp.float32),
            pltpu.VMEM((_BMAX, _N), jnp.float32),
            pltpu.VMEM((_BMAX, _N), jnp.float32),
            pltpu.VMEM((_N, 1), jnp.float32),
            pltpu.SemaphoreType.DMA,
            pltpu.SemaphoreType.DMA,
            pltpu.SemaphoreType.DMA,
            pltpu.SemaphoreType.DMA,
            pltpu.SemaphoreType.DMA,
            pltpu.SemaphoreType.DMA,
        ],
    )(rank, c, sparse_mask)


# speedup vs baseline: 1.0175x; 1.0106x over previous
import jax
import jax.numpy as jnp
from jax.experimental import pallas as pl
from jax.experimental.pallas import tpu as pltpu

_N = 4096
_MAXDEG = 64
_BR = 512


def _row_scale_kernel(rank_ref, c_ref, mask_ref, out_ref):
    i = pl.program_id(0)
    r = rank_ref[pl.ds(i * _BR, _BR)]
    rc = jnp.minimum(r, _MAXDEG - 1)
    bits = [((rc >> b) & 1) == 1 for b in range(6)]
    vals = [c_ref[k] for k in range(_MAXDEG)]
    for b in range(6):
        vals = [jnp.where(bits[b], hi, lo)
                for lo, hi in zip(vals[0::2], vals[1::2])]
    g = vals[0]
    out_ref[...] = g[:, None] * mask_ref[...]


def kernel(x, rank, sparse_mask, c):
    del x
    grid = _N // _BR
    return pl.pallas_call(
        _row_scale_kernel,
        grid=(grid,),
        in_specs=[
            pl.BlockSpec((_N,), lambda i: (0,)),
            pl.BlockSpec(memory_space=pltpu.SMEM),
            pl.BlockSpec((_BR, _N), lambda i: (i, 0)),
        ],
        out_specs=pl.BlockSpec((_BR, _N), lambda i: (i, 0)),
        out_shape=jax.ShapeDtypeStruct((_N, _N), jnp.float32),
        compiler_params=pltpu.CompilerParams(
            dimension_semantics=("arbitrary",),
        ),
    )(rank, c, sparse_mask)
